# baseline (speedup 1.0000x reference)
import jax
import jax.numpy as jnp
from jax import lax
from jax.experimental import pallas as pl
from jax.experimental.pallas import tpu as pltpu

_ALPHA = 12.0
_C_BONDS, _C_AROMA, _C_CHARGE, _C_ELEM = 5, 2, 13, 54
_B, _N = 64, 128

_KEYS = (
    (1832780943, 270669613),
    (64467757, 2916123636),
    (2465931498, 255383827),
    (3134548294, 894150801),
)

_TINY = float(jnp.finfo(jnp.float32).tiny)

_ROT = ((13, 15, 26, 6), (17, 29, 16, 24))


def _threefry_bits(idx, k0, k1):
    k0 = int(k0)
    k1 = int(k1)
    ks2 = (k0 ^ k1 ^ 0x1BD11BDA) & 0xFFFFFFFF
    ks = (k0, k1, ks2)
    x0 = jnp.full(idx.shape, jnp.uint32(k0), jnp.uint32)
    x1 = idx + jnp.uint32(k1)
    for g in range(1, 6):
        for r in _ROT[(g - 1) % 2]:
            x0 = x0 + x1
            x1 = (x1 << r) | (x1 >> (32 - r))
            x1 = x1 ^ x0
        x0 = x0 + jnp.uint32(ks[g % 3])
        x1 = x1 + jnp.uint32((ks[(g + 1) % 3] + g) & 0xFFFFFFFF)
    return x0 ^ x1


def _gumbel(idx, k0, k1):
    bits = _threefry_bits(idx, k0, k1)
    flo = pltpu.bitcast(
        (bits >> 9) | jnp.uint32(0x3F800000), jnp.float32) - jnp.float32(1.0)
    u = jnp.maximum(jnp.float32(_TINY), flo + jnp.float32(_TINY))
    return -jnp.log(-jnp.log(u))


def _sample_classes(pred, curr, init, cf, idx_row, C, k0, k1):
    at, bt, dte, inv1mt, invt = (cf(j) for j in range(5))
    m = pred[0]
    for c in range(1, C):
        m = jnp.maximum(m, pred[c])
    e = [jnp.exp(pred[c] - m) for c in range(C)]
    s = e[0]
    for c in range(1, C):
        s = s + e[c]
    best_val = None
    best_idx = None
    for c in range(C):
        p1c = e[c] / s
        ptc = (curr == c).astype(jnp.float32)
        p0c = (init == c).astype(jnp.float32)
        fwd = inv1mt * (p1c - ptc)
        bwd = invt * (p0c - ptc)
        u = at * fwd - bt * bwd
        prob = jnp.maximum(u * dte + ptc, jnp.float32(1e-10))
        idx_elem = (idx_row * C + c).astype(jnp.uint32)
        score = jnp.log(prob) + _gumbel(idx_elem, k0, k1)
        if c == 0:
            best_val = score
            best_idx = jnp.zeros_like(curr)
        else:
            gt = score > best_val
            best_val = jnp.where(gt, score, best_val)
            best_idx = jnp.where(gt, c, best_idx)
    return best_idx


_WQ = 64


def _bonds_body(pred_ref, curr_ref, init_ref, coef_ref, out_ref):
    i = pl.program_id(0)
    roff = (lax.broadcasted_iota(jnp.int32, (_WQ, 128), 0) * 128
            + lax.broadcasted_iota(jnp.int32, (_WQ, 128), 1))
    idx_row = i * (_WQ * 128) + roff
    cf = lambda j: coef_ref[0, j:j + 1, :]
    pred = [pred_ref[c] for c in range(_C_BONDS)]
    out_ref[0] = _sample_classes(pred, curr_ref[0], init_ref[0], cf, idx_row,
                                 _C_BONDS, *_KEYS[0])


def _small_body(pa_ref, ca_ref, ia_ref, pc_ref, cc_ref, ic_ref,
                pe_ref, ce_ref, ie_ref, coef_ref, oa_ref, oc_ref, oe_ref):
    nq = _B * _N // 128
    roff = (lax.broadcasted_iota(jnp.int32, (nq, 128), 0) * 128
            + lax.broadcasted_iota(jnp.int32, (nq, 128), 1))
    cf = lambda j: coef_ref[j]
    for C, pref, cref, iref, oref, key in (
            (_C_AROMA, pa_ref, ca_ref, ia_ref, oa_ref, _KEYS[1]),
            (_C_CHARGE, pc_ref, cc_ref, ic_ref, oc_ref, _KEYS[2]),
            (_C_ELEM, pe_ref, ce_ref, ie_ref, oe_ref, _KEYS[3])):
        pred = [pref[c] for c in range(C)]
        oref[0] = _sample_classes(pred, cref[0], iref[0], cf, roff, C, *key)


def kernel(curr_bonds, pred_bonds, init_bonds, curr_aromas, pred_aromas,
           init_aromas, curr_charges, pred_charges, init_charges,
           curr_element_types, pred_element_types, init_element_types, t, dt):
    B, N = _B, _N
    M = B * N * N
    Mq = M // 128
    Ms = B * N
    nq = Ms // 128

    at = 1.0 + _ALPHA * t ** 2.0 * (1.0 - t) ** 0.5
    bt = at - 1.0
    alpha_term = at * 1.0 / (1.0 - t)
    beta_term = bt * 1.0 / t
    dte = jnp.minimum(dt, 1.0 / (alpha_term + beta_term))
    inv1mt = 1.0 / (1.0 - t)
    invt = 1.0 / t
    coef = jnp.stack([at, bt, dte, inv1mt, invt], axis=0)

    coef_bonds = jnp.broadcast_to(coef.T[:, :, None], (B, 5, 128))
    coef_small = jnp.broadcast_to(coef[:, :, None], (5, B, 128))

    pred_b = pred_bonds.reshape(M, _C_BONDS).T.reshape(_C_BONDS, Mq, 128)
    curr_b = curr_bonds.reshape(1, Mq, 128)
    init_b = init_bonds.reshape(1, Mq, 128)

    grid = Mq // _WQ
    blk3 = lambda C: pl.BlockSpec((C, _WQ, 128), lambda i: (0, i, 0))
    out_bonds = pl.pallas_call(
        _bonds_body,
        grid=(grid,),
        in_specs=[
            blk3(_C_BONDS),
            blk3(1),
            blk3(1),
            pl.BlockSpec((1, 5, 128), lambda i: (i // 2, 0, 0)),
        ],
        out_specs=blk3(1),
        out_shape=jax.ShapeDtypeStruct((1, Mq, 128), jnp.int32),
    )(pred_b, curr_b, init_b, coef_bonds)

    def prep(pred, C):
        return pred.reshape(Ms, C).T.reshape(C, nq, 128)

    full = lambda shape: pl.BlockSpec(shape, lambda: (0, 0, 0))
    small_out = jax.ShapeDtypeStruct((1, nq, 128), jnp.int32)
    out_a, out_c, out_e = pl.pallas_call(
        _small_body,
        in_specs=[
            full((_C_AROMA, nq, 128)), full((1, nq, 128)), full((1, nq, 128)),
            full((_C_CHARGE, nq, 128)), full((1, nq, 128)), full((1, nq, 128)),
            full((_C_ELEM, nq, 128)), full((1, nq, 128)), full((1, nq, 128)),
            full((5, nq, 128)),
        ],
        out_specs=[full((1, nq, 128))] * 3,
        out_shape=[small_out] * 3,
    )(prep(pred_aromas, _C_AROMA), curr_aromas.reshape(1, nq, 128),
      init_aromas.reshape(1, nq, 128),
      prep(pred_charges, _C_CHARGE), curr_charges.reshape(1, nq, 128),
      init_charges.reshape(1, nq, 128),
      prep(pred_element_types, _C_ELEM),
      curr_element_types.reshape(1, nq, 128),
      init_element_types.reshape(1, nq, 128),
      coef_small)

    return (out_bonds.reshape(B, N, N), out_a.reshape(B, N),
            out_c.reshape(B, N), out_e.reshape(B, N))
